# baseline (speedup 1.0000x reference)
import jax
import jax.numpy as jnp
from jax.experimental import pallas as pl
from jax.experimental.pallas import tpu as pltpu

N = 4096
D = 256
V = 3
H = 64
B = 1024
A = 128
C = 4
SLOPE = 0.2

RPB = 512
NB_B = B // RPB
NSTEPS = V * NB_B
NBUF = 3


def _fused_kernel(idx_ref, thr_ref, col_ref, feat_ref, wi_ref, bi_ref,
                  wa_ref, ba_ref, va_ref, wo_ref, bo_ref, w_ref,
                  bf_ref, pred_ref, buf, sems, hmain_buf, hall_buf,
                  hg_buf, stack_buf, e_smem):
    v = pl.program_id(0)
    b = pl.program_id(1)
    step = v * NB_B + b

    def issue(t, slot):
        tv = t // NB_B
        tb = t % NB_B
        for j in range(RPB):
            row = tv * N + idx_ref[tb * RPB + j]
            pltpu.make_async_copy(
                w_ref.at[pl.ds(row, 1), :],
                buf.at[slot, pl.ds(j, 1), :],
                sems.at[slot],
            ).start()

    @pl.when(step == 0)
    def _():
        for t in range(NBUF - 1):
            issue(t, t % NBUF)

    @pl.when(step + NBUF - 1 < NSTEPS)
    def _():
        issue(step + NBUF - 1, (step + NBUF - 1) % NBUF)

    @pl.when(step == 0)
    def _():
        lane = jax.lax.broadcasted_iota(jnp.int32, (N, 2 * H), 1)
        phs = []
        for i in range(V):
            ph = (
                jnp.dot(feat_ref[...], wi_ref[i],
                        preferred_element_type=jnp.float32)
                + bi_ref[i]
            ).astype(jnp.bfloat16)
            phs.append(ph)
            padded = jnp.concatenate(
                [ph, jnp.zeros((N, H), jnp.bfloat16)], axis=1)
            hmain_buf[i] = jnp.where(lane == H, jnp.bfloat16(1.0), padded)
        hall_buf[...] = jnp.concatenate(phs, axis=1)

    @pl.when(v == 0)
    def _():
        col = col_ref[0]
        iota = jax.lax.broadcasted_iota(jnp.int32, (RPB, N), 1)
        oh = (iota == col).astype(jnp.bfloat16)
        hg = jnp.dot(oh, hall_buf[...], preferred_element_type=jnp.float32)
        for i in range(V):
            hg_buf[i, pl.ds(b * RPB, RPB), :] = hg[:, i * H:(i + 1) * H]

    slot = jax.lax.rem(step, NBUF)
    pltpu.make_async_copy(
        w_ref.at[pl.ds(0, RPB), :],
        buf.at[slot],
        sems.at[slot],
    ).wait()

    rows_bf = buf[slot].astype(jnp.bfloat16)
    thr = thr_ref[0, 0, 0].astype(jnp.bfloat16)
    aug = jnp.where(rows_bf >= thr, rows_bf, jnp.bfloat16(0.0))
    acc = jnp.dot(aug, hmain_buf[v], preferred_element_type=jnp.float32)
    deg = jnp.maximum(acc[:, H:H + 1], 1e-12)
    res = acc[:, :H] / deg + hg_buf[v, pl.ds(b * RPB, RPB), :]
    stack_buf[v, pl.ds(b * RPB, RPB), :] = jnp.where(res >= 0, res, SLOPE * res)

    def view_e(stk_v):
        s = jnp.tanh(
            jnp.dot(stk_v, wa_ref[...], preferred_element_type=jnp.float32)
            + ba_ref[...]
        )
        return jnp.sum(s * va_ref[...]) / B

    @pl.when((b == 0) & (v >= 1))
    def _():
        e_smem[v - 1] = view_e(stack_buf[v - 1])

    @pl.when(step == NSTEPS - 1)
    def _():
        stk = stack_buf[...]
        es = [e_smem[0], e_smem[1], view_e(stk[V - 1])]
        m = jnp.maximum(es[0], jnp.maximum(es[1], es[2]))
        ws = [jnp.exp(e - m) for e in es]
        tot = ws[0] + ws[1] + ws[2]
        bf = (ws[0] * stk[0] + ws[1] * stk[1] + ws[2] * stk[2]) / tot
        bf_ref[...] = bf
        pred_ref[...] = (
            jnp.dot(bf, wo_ref[...], preferred_element_type=jnp.float32)
            + bo_ref[...]
        )


@jax.jit
def kernel(features, weights, batch_idx, thresholds, W_intra, b_intra,
           W_attn, b_attn, v_attn, W_out, b_out):
    batch_idx = batch_idx.astype(jnp.int32)
    weights_2d = weights.reshape(V * N, N)
    thr_r = thresholds.reshape(V, 1, 1)
    col_r = batch_idx.reshape(NB_B, RPB, 1)

    grid_spec = pltpu.PrefetchScalarGridSpec(
        num_scalar_prefetch=1,
        grid=(V, NB_B),
        in_specs=[
            pl.BlockSpec((1, 1, 1), lambda v, b, idx: (v, 0, 0)),
            pl.BlockSpec((1, RPB, 1), lambda v, b, idx: (b, 0, 0)),
            pl.BlockSpec((N, D), lambda v, b, idx: (0, 0)),
            pl.BlockSpec((V, D, H), lambda v, b, idx: (0, 0, 0)),
            pl.BlockSpec((V, 1, H), lambda v, b, idx: (0, 0, 0)),
            pl.BlockSpec((H, A), lambda v, b, idx: (0, 0)),
            pl.BlockSpec((1, A), lambda v, b, idx: (0, 0)),
            pl.BlockSpec((1, A), lambda v, b, idx: (0, 0)),
            pl.BlockSpec((H, C), lambda v, b, idx: (0, 0)),
            pl.BlockSpec((1, C), lambda v, b, idx: (0, 0)),
            pl.BlockSpec(memory_space=pl.ANY),
        ],
        out_specs=[
            pl.BlockSpec((B, H), lambda v, b, idx: (0, 0)),
            pl.BlockSpec((B, C), lambda v, b, idx: (0, 0)),
        ],
        scratch_shapes=[
            pltpu.VMEM((NBUF, RPB, N), jnp.float32),
            pltpu.SemaphoreType.DMA((NBUF,)),
            pltpu.VMEM((V, N, 2 * H), jnp.bfloat16),
            pltpu.VMEM((N, V * H), jnp.bfloat16),
            pltpu.VMEM((V, B, H), jnp.float32),
            pltpu.VMEM((V, B, H), jnp.float32),
            pltpu.SMEM((8,), jnp.float32),
        ],
    )
    bf, pred = pl.pallas_call(
        _fused_kernel,
        grid_spec=grid_spec,
        out_shape=[
            jax.ShapeDtypeStruct((B, H), jnp.float32),
            jax.ShapeDtypeStruct((B, C), jnp.float32),
        ],
    )(batch_idx, thr_r, col_r, features, W_intra, b_intra.reshape(V, 1, H),
      W_attn, b_attn.reshape(1, A), v_attn.reshape(1, A),
      W_out, b_out.reshape(1, C), weights_2d)

    return (bf, pred)


# speedup vs baseline: 1.0177x; 1.0177x over previous
import jax
import jax.numpy as jnp
from jax.experimental import pallas as pl
from jax.experimental.pallas import tpu as pltpu

N = 4096
D = 256
V = 3
H = 64
B = 1024
A = 128
C = 4
SLOPE = 0.2

RPB = 512
NB_B = B // RPB
NSTEPS = V * NB_B
NBUF = 3


def _fused_kernel(idx_ref, thr_ref, col_ref, feat_ref, wi_ref, bi_ref,
                  wa_ref, ba_ref, va_ref, wo_ref, bo_ref, w_ref,
                  bf_ref, pred_ref, buf, sems, hmain_buf, hall_buf,
                  hg_buf, stack_buf):
    v = pl.program_id(0)
    b = pl.program_id(1)
    step = v * NB_B + b

    def issue(t, slot):
        tv = t // NB_B
        tb = t % NB_B
        for j in range(RPB):
            row = tv * N + idx_ref[tb * RPB + j]
            pltpu.make_async_copy(
                w_ref.at[pl.ds(row, 1), :],
                buf.at[slot, pl.ds(j, 1), :],
                sems.at[slot],
            ).start()

    @pl.when(step == 0)
    def _():
        for t in range(NBUF - 1):
            issue(t, t % NBUF)

    @pl.when(step + NBUF - 1 < NSTEPS)
    def _():
        issue(step + NBUF - 1, (step + NBUF - 1) % NBUF)

    @pl.when(step == 0)
    def _():
        lane = jax.lax.broadcasted_iota(jnp.int32, (N, 2 * H), 1)
        phs = []
        for i in range(V):
            ph = (
                jnp.dot(feat_ref[...], wi_ref[i],
                        preferred_element_type=jnp.float32)
                + bi_ref[i]
            ).astype(jnp.bfloat16)
            phs.append(ph)
            padded = jnp.concatenate(
                [ph, jnp.zeros((N, H), jnp.bfloat16)], axis=1)
            hmain_buf[i] = jnp.where(lane == H, jnp.bfloat16(1.0), padded)
        hall_buf[...] = jnp.concatenate(phs, axis=1)

    @pl.when(v == 0)
    def _():
        col = col_ref[0]
        iota = jax.lax.broadcasted_iota(jnp.int32, (RPB, N), 1)
        oh = (iota == col).astype(jnp.bfloat16)
        hg = jnp.dot(oh, hall_buf[...], preferred_element_type=jnp.float32)
        for i in range(V):
            hg_buf[i, pl.ds(b * RPB, RPB), :] = hg[:, i * H:(i + 1) * H]

    slot = jax.lax.rem(step, NBUF)
    pltpu.make_async_copy(
        w_ref.at[pl.ds(0, RPB), :],
        buf.at[slot],
        sems.at[slot],
    ).wait()

    rows_bf = buf[slot].astype(jnp.bfloat16)
    thr = thr_ref[0, 0, 0].astype(jnp.bfloat16)
    aug = jnp.where(rows_bf >= thr, rows_bf, jnp.bfloat16(0.0))
    acc = jnp.dot(aug, hmain_buf[v], preferred_element_type=jnp.float32)
    deg = jnp.maximum(acc[:, H:H + 1], 1e-12)
    res = acc[:, :H] / deg + hg_buf[v, pl.ds(b * RPB, RPB), :]
    stack_buf[v, pl.ds(b * RPB, RPB), :] = jnp.where(res >= 0, res, SLOPE * res)

    @pl.when(step == NSTEPS - 1)
    def _():
        stk = stack_buf[...]
        wa = wa_ref[...]
        ba = ba_ref[...]
        va = va_ref[...]
        es = []
        for i in range(V):
            s = jnp.tanh(
                jnp.dot(stk[i], wa, preferred_element_type=jnp.float32) + ba
            )
            es.append(jnp.sum(s * va) / B)
        m = jnp.maximum(es[0], jnp.maximum(es[1], es[2]))
        ws = [jnp.exp(e - m) for e in es]
        tot = ws[0] + ws[1] + ws[2]
        bf = (ws[0] * stk[0] + ws[1] * stk[1] + ws[2] * stk[2]) / tot
        bf_ref[...] = bf
        pred_ref[...] = (
            jnp.dot(bf, wo_ref[...], preferred_element_type=jnp.float32)
            + bo_ref[...]
        )


@jax.jit
def kernel(features, weights, batch_idx, thresholds, W_intra, b_intra,
           W_attn, b_attn, v_attn, W_out, b_out):
    batch_idx = batch_idx.astype(jnp.int32)
    weights_2d = weights.reshape(V * N, N)
    thr_r = thresholds.reshape(V, 1, 1)
    col_r = batch_idx.reshape(NB_B, RPB, 1)

    grid_spec = pltpu.PrefetchScalarGridSpec(
        num_scalar_prefetch=1,
        grid=(V, NB_B),
        in_specs=[
            pl.BlockSpec((1, 1, 1), lambda v, b, idx: (v, 0, 0)),
            pl.BlockSpec((1, RPB, 1), lambda v, b, idx: (b, 0, 0)),
            pl.BlockSpec((N, D), lambda v, b, idx: (0, 0)),
            pl.BlockSpec((V, D, H), lambda v, b, idx: (0, 0, 0)),
            pl.BlockSpec((V, 1, H), lambda v, b, idx: (0, 0, 0)),
            pl.BlockSpec((H, A), lambda v, b, idx: (0, 0)),
            pl.BlockSpec((1, A), lambda v, b, idx: (0, 0)),
            pl.BlockSpec((1, A), lambda v, b, idx: (0, 0)),
            pl.BlockSpec((H, C), lambda v, b, idx: (0, 0)),
            pl.BlockSpec((1, C), lambda v, b, idx: (0, 0)),
            pl.BlockSpec(memory_space=pl.ANY),
        ],
        out_specs=[
            pl.BlockSpec((B, H), lambda v, b, idx: (0, 0)),
            pl.BlockSpec((B, C), lambda v, b, idx: (0, 0)),
        ],
        scratch_shapes=[
            pltpu.VMEM((NBUF, RPB, N), jnp.float32),
            pltpu.SemaphoreType.DMA((NBUF,)),
            pltpu.VMEM((V, N, 2 * H), jnp.bfloat16),
            pltpu.VMEM((N, V * H), jnp.bfloat16),
            pltpu.VMEM((V, B, H), jnp.float32),
            pltpu.VMEM((V, B, H), jnp.float32),
        ],
    )
    bf, pred = pl.pallas_call(
        _fused_kernel,
        grid_spec=grid_spec,
        out_shape=[
            jax.ShapeDtypeStruct((B, H), jnp.float32),
            jax.ShapeDtypeStruct((B, C), jnp.float32),
        ],
    )(batch_idx, thr_r, col_r, features, W_intra, b_intra.reshape(V, 1, H),
      W_attn, b_attn.reshape(1, A), v_attn.reshape(1, A),
      W_out, b_out.reshape(1, C), weights_2d)

    return (bf, pred)
